# baseline (speedup 1.0000x reference)
import functools

import jax
import jax.numpy as jnp
from jax import lax
from jax.experimental import pallas as pl
from jax.experimental.pallas import tpu as pltpu
from jax.experimental.pallas import tpu_sc as plsc

_CHUNK = 128


@functools.lru_cache(maxsize=None)
def _build(B, V, D):
    info = plsc.get_sparse_core_info()
    NC, NS = info.num_cores, info.num_subcores
    NW = NC * NS
    b_per_w = B // NW
    n_chunk = b_per_w // _CHUNK

    mesh = plsc.VectorSubcoreMesh(core_axis_name="c", subcore_axis_name="s")

    @functools.partial(
        pl.kernel,
        mesh=mesh,
        out_type=jax.ShapeDtypeStruct((B, D), jnp.float32),
        compiler_params=pltpu.CompilerParams(use_tc_tiling_on_sc=False),
        scratch_types=[
            pltpu.VMEM((n_chunk, _CHUNK), jnp.int32),
            pltpu.VMEM((b_per_w, D), jnp.float32),
            pltpu.SemaphoreType.DMA,
        ],
    )
    def gather_kernel(idx_hbm, table_hbm, out_hbm, idx_v, rows_v, sem):
        wid = lax.axis_index("s") * NC + lax.axis_index("c")
        base = wid * b_per_w
        pltpu.sync_copy(idx_hbm.at[wid], idx_v)
        copies = [
            pltpu.async_copy(
                table_hbm.at[idx_v.at[j]],
                rows_v.at[pl.ds(j * _CHUNK, _CHUNK)],
                sem,
            )
            for j in range(n_chunk)
        ]
        for c in copies:
            c.wait()
        pltpu.sync_copy(rows_v, out_hbm.at[pl.ds(base, b_per_w)])

    return gather_kernel, NW, n_chunk


def kernel(t, emb_weight):
    (B,) = t.shape
    V, D = emb_weight.shape
    fn, NW, n_chunk = _build(B, V, D)
    idx = t.astype(jnp.int32).reshape(NW, n_chunk, _CHUNK)
    return fn(idx, emb_weight)


# speedup vs baseline: 1.0011x; 1.0011x over previous
import functools

import jax
import jax.numpy as jnp
from jax import lax
from jax.experimental import pallas as pl
from jax.experimental.pallas import tpu as pltpu
from jax.experimental.pallas import tpu_sc as plsc

_CHUNK = 128


@functools.lru_cache(maxsize=None)
def _build(B, V, D):
    info = plsc.get_sparse_core_info()
    NC, NS = info.num_cores, info.num_subcores
    NW = NC * NS
    b_per_w = B // NW
    n_chunk = b_per_w // _CHUNK

    mesh = plsc.VectorSubcoreMesh(core_axis_name="c", subcore_axis_name="s")

    @functools.partial(
        pl.kernel,
        mesh=mesh,
        out_type=jax.ShapeDtypeStruct((B, D), jnp.float32),
        compiler_params=pltpu.CompilerParams(use_tc_tiling_on_sc=False),
        scratch_types=[
            pltpu.VMEM((b_per_w,), jnp.int32),
            pltpu.VMEM((b_per_w, D), jnp.float32),
            pltpu.SemaphoreType.DMA,
        ],
    )
    def gather_kernel(idx_hbm, table_hbm, out_hbm, idx_v, rows_v, sem):
        wid = lax.axis_index("s") * NC + lax.axis_index("c")
        base = wid * b_per_w
        pltpu.sync_copy(idx_hbm.at[pl.ds(base, b_per_w)], idx_v)
        copies = [
            pltpu.async_copy(
                table_hbm.at[idx_v.at[pl.ds(j * _CHUNK, _CHUNK)]],
                rows_v.at[pl.ds(j * _CHUNK, _CHUNK)],
                sem,
            )
            for j in range(n_chunk)
        ]
        for c in copies:
            c.wait()
        pltpu.sync_copy(rows_v, out_hbm.at[pl.ds(base, b_per_w)])

    return gather_kernel, NW, n_chunk


def kernel(t, emb_weight):
    (B,) = t.shape
    V, D = emb_weight.shape
    fn, NW, n_chunk = _build(B, V, D)
    return fn(t.astype(jnp.int32), emb_weight)


# speedup vs baseline: 1.0035x; 1.0024x over previous
import functools

import jax
import jax.numpy as jnp
from jax import lax
from jax.experimental import pallas as pl
from jax.experimental.pallas import tpu as pltpu
from jax.experimental.pallas import tpu_sc as plsc

_CHUNK = 128


@functools.lru_cache(maxsize=None)
def _build(B, V, D):
    info = plsc.get_sparse_core_info()
    NC, NS = info.num_cores, info.num_subcores
    NW = NC * NS
    b_per_w = B // NW
    n_chunk = b_per_w // _CHUNK

    mesh = plsc.VectorSubcoreMesh(core_axis_name="c", subcore_axis_name="s")

    @functools.partial(
        pl.kernel,
        mesh=mesh,
        out_type=jax.ShapeDtypeStruct((B, D), jnp.float32),
        compiler_params=pltpu.CompilerParams(use_tc_tiling_on_sc=False),
        scratch_types=[
            pltpu.VMEM((b_per_w,), jnp.int32),
            pltpu.VMEM((b_per_w, D), jnp.float32),
            pltpu.SemaphoreType.DMA,
        ],
    )
    def gather_kernel(idx_hbm, table_hbm, out_hbm, idx_v, rows_v, sem):
        wid = lax.axis_index("s") * NC + lax.axis_index("c")
        base = wid * b_per_w
        pltpu.sync_copy(idx_hbm.at[pl.ds(base, b_per_w)], idx_v)
        copies = [
            pltpu.async_copy(
                table_hbm.at[idx_v.at[pl.ds(j * _CHUNK, _CHUNK)]],
                rows_v.at[pl.ds(j * _CHUNK, _CHUNK)],
                sem,
            )
            for j in range(n_chunk)
        ]
        for c in copies:
            c.wait()
        pltpu.sync_copy(rows_v, out_hbm.at[pl.ds(base, b_per_w)])

    return gather_kernel, NW


def kernel(t, emb_weight):
    (B,) = t.shape
    V, D = emb_weight.shape
    fn, NW = _build(B, V, D)
    return fn(t.astype(jnp.int32), emb_weight)


# speedup vs baseline: 1.3575x; 1.3527x over previous
import functools

import jax
import jax.numpy as jnp
from jax import lax
from jax.experimental import pallas as pl
from jax.experimental.pallas import tpu as pltpu
from jax.experimental.pallas import tpu_sc as plsc

_WAVE = 64


@functools.lru_cache(maxsize=None)
def _build(B, V, D):
    info = plsc.get_sparse_core_info()
    NC, NS = info.num_cores, info.num_subcores
    NW = NC * NS
    b_per_w = B // NW

    mesh = plsc.VectorSubcoreMesh(core_axis_name="c", subcore_axis_name="s")

    @functools.partial(
        pl.kernel,
        mesh=mesh,
        out_type=jax.ShapeDtypeStruct((B // 8, 8, D), jnp.float32),
        scratch_types=[
            pltpu.VMEM((b_per_w,), jnp.int32),
            pltpu.VMEM((b_per_w // 8, 8, D), jnp.float32),
            pltpu.SemaphoreType.DMA,
        ],
    )
    def gather_kernel(idx_hbm, table_hbm, out_hbm, idx_v, rows_v, sem):
        wid = lax.axis_index("s") * NC + lax.axis_index("c")
        base = wid * b_per_w
        pltpu.sync_copy(idx_hbm.at[pl.ds(base, b_per_w)], idx_v)

        def fire(g, carry):
            vec = idx_v[pl.ds(g * 16, 16)]
            for l in range(16):
                pltpu.async_copy(
                    table_hbm.at[vec[l]],
                    rows_v.at[2 * g + l // 8, l % 8],
                    sem,
                )
            return carry

        def drain(i, carry):
            pltpu.make_async_copy(
                table_hbm.at[0], rows_v.at[i >> 3, i & 7], sem
            ).wait()
            return carry

        for w in range(b_per_w // _WAVE):
            lax.fori_loop(
                w * _WAVE // 16, (w + 1) * _WAVE // 16, fire, 0
            )
            lax.fori_loop(w * _WAVE, (w + 1) * _WAVE, drain, 0)

        pltpu.sync_copy(rows_v, out_hbm.at[pl.ds(base // 8, b_per_w // 8)])

    return gather_kernel, NW


def kernel(t, emb_weight):
    (B,) = t.shape
    V, D = emb_weight.shape
    fn, NW = _build(B, V, D)
    out3 = fn(t.astype(jnp.int32), emb_weight)
    return out3.reshape(B, D)


# speedup vs baseline: 1.4318x; 1.0548x over previous
import functools

import jax
import jax.numpy as jnp
from jax import lax
from jax.experimental import pallas as pl
from jax.experimental.pallas import tpu as pltpu
from jax.experimental.pallas import tpu_sc as plsc

_WAVE = 64


@functools.lru_cache(maxsize=None)
def _build(B, V, D):
    info = plsc.get_sparse_core_info()
    NC, NS = info.num_cores, info.num_subcores
    NW = NC * NS
    b_per_w = B // NW

    mesh = plsc.VectorSubcoreMesh(core_axis_name="c", subcore_axis_name="s")

    @functools.partial(
        pl.kernel,
        mesh=mesh,
        out_type=jax.ShapeDtypeStruct((B // 8, 8, D), jnp.float32),
        scratch_types=[
            pltpu.VMEM((b_per_w,), jnp.int32),
            pltpu.VMEM((b_per_w // 8, 8, D), jnp.float32),
            pltpu.SemaphoreType.DMA,
        ],
    )
    def gather_kernel(idx_hbm, table_hbm, out_hbm, idx_v, rows_v, sem):
        wid = lax.axis_index("s") * NC + lax.axis_index("c")
        base = wid * b_per_w
        pltpu.sync_copy(idx_hbm.at[pl.ds(base, b_per_w)], idx_v)

        def fire(g, carry):
            vec = idx_v[pl.ds(g * 16, 16)]
            for l in range(16):
                pltpu.async_copy(
                    table_hbm.at[vec[l]],
                    rows_v.at[2 * g + l // 8, l % 8],
                    sem,
                )
            return carry

        def drain(i, carry):
            pltpu.make_async_copy(
                table_hbm.at[0], rows_v.at[i >> 3, i & 7], sem
            ).wait()
            return carry

        n_wave = b_per_w // _WAVE
        for w in range(n_wave):
            lax.fori_loop(
                w * _WAVE // 16, (w + 1) * _WAVE // 16, fire, 0
            )
            if w > 0:
                lax.fori_loop((w - 1) * _WAVE, w * _WAVE, drain, 0)
        lax.fori_loop((n_wave - 1) * _WAVE, n_wave * _WAVE, drain, 0)

        pltpu.sync_copy(rows_v, out_hbm.at[pl.ds(base // 8, b_per_w // 8)])

    return gather_kernel, NW


def kernel(t, emb_weight):
    (B,) = t.shape
    V, D = emb_weight.shape
    fn, NW = _build(B, V, D)
    out3 = fn(t.astype(jnp.int32), emb_weight)
    return out3.reshape(B, D)


# speedup vs baseline: 1.4385x; 1.0047x over previous
import functools

import jax
import jax.numpy as jnp
from jax import lax
from jax.experimental import pallas as pl
from jax.experimental.pallas import tpu as pltpu
from jax.experimental.pallas import tpu_sc as plsc

_WAVE = 64
_L = 16


@functools.lru_cache(maxsize=None)
def _build(B, V, D):
    info = plsc.get_sparse_core_info()
    NC, NS = info.num_cores, info.num_subcores
    NW = NC * NS
    b_per_w = B // NW

    mesh = plsc.VectorSubcoreMesh(core_axis_name="c", subcore_axis_name="s")

    @functools.partial(
        pl.kernel,
        mesh=mesh,
        out_type=jax.ShapeDtypeStruct((B // 8, 8, D), jnp.float32),
        scratch_types=[
            pltpu.VMEM((b_per_w,), jnp.int32),
            pltpu.VMEM((b_per_w // 8, 8, D), jnp.float32),
            pltpu.SemaphoreType.DMA,
        ],
    )
    def gather_kernel(idx_hbm, table_hbm, out_hbm, idx_v, rows_v, sem):
        wid = lax.axis_index("s") * NC + lax.axis_index("c")
        base = wid * b_per_w
        pltpu.sync_copy(idx_hbm.at[pl.ds(base, b_per_w)], idx_v)

        def fire(g, carry):
            vec = idx_v[pl.ds(g * _L, _L)]
            for l in range(_L):
                pltpu.async_copy(
                    table_hbm.at[vec[l]],
                    rows_v.at[2 * g + l // 8, l % 8],
                    sem,
                )
            return carry

        def drain_wave():
            pltpu.make_async_copy(
                out_hbm.at[pl.ds(0, _WAVE // 8)],
                rows_v.at[pl.ds(0, _WAVE // 8)],
                sem,
            ).wait()

        n_wave = b_per_w // _WAVE
        for w in range(n_wave):
            lax.fori_loop(w * _WAVE // _L, (w + 1) * _WAVE // _L, fire, 0)
            if w > 0:
                drain_wave()
        drain_wave()

        pltpu.sync_copy(rows_v, out_hbm.at[pl.ds(base // 8, b_per_w // 8)])

    return gather_kernel, NW


def kernel(t, emb_weight):
    (B,) = t.shape
    V, D = emb_weight.shape
    fn, NW = _build(B, V, D)
    out3 = fn(t.astype(jnp.int32), emb_weight)
    return out3.reshape(B, D)


# speedup vs baseline: 1.6458x; 1.1441x over previous
import functools

import jax
import jax.numpy as jnp
from jax import lax
from jax.experimental import pallas as pl
from jax.experimental.pallas import tpu as pltpu
from jax.experimental.pallas import tpu_sc as plsc

_WAVE = 64
_L = 16


@functools.lru_cache(maxsize=None)
def _build(B, V, D):
    info = plsc.get_sparse_core_info()
    NC, NS = info.num_cores, info.num_subcores
    NW = NC * NS
    b_per_w = B // NW

    mesh = plsc.VectorSubcoreMesh(core_axis_name="c", subcore_axis_name="s")

    @functools.partial(
        pl.kernel,
        mesh=mesh,
        out_type=jax.ShapeDtypeStruct((B // 8, 8, D), jnp.float32),
        scratch_types=[
            pltpu.VMEM((b_per_w,), jnp.int32),
            pltpu.VMEM((b_per_w // 8, 8, D), jnp.float32),
            pltpu.SemaphoreType.DMA,
        ],
    )
    def gather_kernel(idx_hbm, table_hbm, out_hbm, idx_v, rows_v, sem):
        wid = lax.axis_index("s") * NC + lax.axis_index("c")
        base = wid * b_per_w
        pltpu.sync_copy(idx_hbm.at[pl.ds(base, b_per_w)], idx_v)

        def fire(g, carry):
            vec = idx_v[pl.ds(g * _L, _L)]
            for l in range(_L):
                v = vec[l]
                pltpu.async_copy(
                    table_hbm.at[v >> 3, v & 7],
                    rows_v.at[2 * g + l // 8, l % 8],
                    sem,
                )
            return carry

        def drain_wave():
            pltpu.make_async_copy(
                out_hbm.at[pl.ds(0, _WAVE // 8)],
                rows_v.at[pl.ds(0, _WAVE // 8)],
                sem,
            ).wait()

        n_wave = b_per_w // _WAVE
        for w in range(n_wave):
            lax.fori_loop(w * _WAVE // _L, (w + 1) * _WAVE // _L, fire, 0)
            if w > 0:
                drain_wave()
        drain_wave()

        pltpu.sync_copy(rows_v, out_hbm.at[pl.ds(base // 8, b_per_w // 8)])

    return gather_kernel, NW


def kernel(t, emb_weight):
    (B,) = t.shape
    V, D = emb_weight.shape
    fn, NW = _build(B, V, D)
    out3 = fn(t.astype(jnp.int32), emb_weight.reshape(V // 8, 8, D))
    return out3.reshape(B, D)


# speedup vs baseline: 1.7045x; 1.0357x over previous
import functools

import jax
import jax.numpy as jnp
from jax import lax
from jax.experimental import pallas as pl
from jax.experimental.pallas import tpu as pltpu
from jax.experimental.pallas import tpu_sc as plsc

_WAVE = 128
_L = 16


@functools.lru_cache(maxsize=None)
def _build(B, V, D):
    info = plsc.get_sparse_core_info()
    NC, NS = info.num_cores, info.num_subcores
    NW = NC * NS
    b_per_w = B // NW

    mesh = plsc.VectorSubcoreMesh(core_axis_name="c", subcore_axis_name="s")

    @functools.partial(
        pl.kernel,
        mesh=mesh,
        out_type=jax.ShapeDtypeStruct((B // 8, 8, D), jnp.float32),
        scratch_types=[
            pltpu.VMEM((b_per_w,), jnp.int32),
            pltpu.VMEM((b_per_w // 8, 8, D), jnp.float32),
            pltpu.SemaphoreType.DMA,
        ],
    )
    def gather_kernel(idx_hbm, table_hbm, out_hbm, idx_v, rows_v, sem):
        wid = lax.axis_index("s") * NC + lax.axis_index("c")
        base = wid * b_per_w
        pltpu.sync_copy(idx_hbm.at[pl.ds(base, b_per_w)], idx_v)

        def fire(g, carry):
            vec = idx_v[pl.ds(g * _L, _L)]
            bv = vec >> 3
            sv = vec & 7
            for l in range(_L):
                pltpu.async_copy(
                    table_hbm.at[bv[l], sv[l]],
                    rows_v.at[2 * g + l // 8, l % 8],
                    sem,
                )
            return carry

        def drain_wave():
            pltpu.make_async_copy(
                out_hbm.at[pl.ds(0, _WAVE // 8)],
                rows_v.at[pl.ds(0, _WAVE // 8)],
                sem,
            ).wait()

        n_wave = b_per_w // _WAVE
        for w in range(n_wave):
            lax.fori_loop(w * _WAVE // _L, (w + 1) * _WAVE // _L, fire, 0)
            if w > 0:
                drain_wave()
        drain_wave()

        pltpu.sync_copy(rows_v, out_hbm.at[pl.ds(base // 8, b_per_w // 8)])

    return gather_kernel, NW


def kernel(t, emb_weight):
    (B,) = t.shape
    V, D = emb_weight.shape
    fn, NW = _build(B, V, D)
    out3 = fn(t.astype(jnp.int32), emb_weight.reshape(V // 8, 8, D))
    return out3.reshape(B, D)
